# Initial kernel scaffold; baseline (speedup 1.0000x reference)
import jax
import jax.numpy as jnp
from jax.experimental import pallas as pl


def kernel(x, edge_index, sage_Wl, sage_Wr, sage_b, gn_gamma, gn_beta, gn_alpha, dyt_alpha, dyt_gamma, dyt_beta, lin1_W, lin1_b, lin2_W, lin2_b, lin3_W, lin3_b, jv1_W, jv1_b, jv2_W, jv2_b):
    raise NotImplementedError("write your pallas kernel here")


# speedup vs baseline: 4.6963x; 4.6963x over previous
import functools

import jax
import jax.numpy as jnp
from jax import lax
from jax.experimental import pallas as pl
from jax.experimental.pallas import tpu as pltpu
from jax.experimental.pallas import tpu_sc as plsc

N = 10000
E = 320000
D = 128
L = 3

NC = 2
NS = 16
NW = NC * NS
EPW = E // NW
K = 80
NCHUNK = EPW // K
RPT = 624
RTAIL = N - NS * RPT

_f32 = jnp.float32

_MESH = plsc.VectorSubcoreMesh(core_axis_name="c", subcore_axis_name="s")


def _zero_spmem(s, z_nd, sh):
  pltpu.sync_copy(z_nd.at[pl.ds(s * RPT, RPT)], sh.at[pl.ds(s * RPT, RPT)])

  @pl.when(s == 0)
  def _():
    pltpu.sync_copy(z_nd.at[pl.ds(NS * RPT, RTAIL)],
                    sh.at[pl.ds(NS * RPT, RTAIL)])


def _flush_spmem(c, s, sh, out):
  pltpu.sync_copy(sh.at[pl.ds(s * RPT, RPT)], out.at[c, pl.ds(s * RPT, RPT)])

  @pl.when(s == 0)
  def _():
    pltpu.sync_copy(sh.at[pl.ds(NS * RPT, RTAIL)],
                    out.at[c, pl.ds(NS * RPT, RTAIL)])


@functools.partial(
    pl.kernel, mesh=_MESH,
    out_type=jax.ShapeDtypeStruct((NC, N, D), _f32),
    scratch_types=[
        pltpu.VMEM((K,), jnp.int32),
        pltpu.VMEM((K,), jnp.int32),
        pltpu.VMEM((K, D), _f32),
        pltpu.VMEM_SHARED((N, D), _f32),
        pltpu.SemaphoreType.DMA,
    ])
def _sc_agg(h_hbm, src_hbm, dst_hbm, z_nd, out_agg,
            sidx, didx, rows, agg_sh, sem):
  c = lax.axis_index("c")
  s = lax.axis_index("s")
  wid = c * NS + s

  _zero_spmem(s, z_nd, agg_sh)
  plsc.subcore_barrier()

  ebase = wid * EPW

  def chunk(j, carry):
    base = ebase + j * K
    pltpu.sync_copy(src_hbm.at[pl.ds(base, K)], sidx)
    pltpu.sync_copy(dst_hbm.at[pl.ds(base, K)], didx)
    pltpu.async_copy(h_hbm.at[sidx], rows, sem).wait()
    pltpu.sync_copy(rows, agg_sh.at[didx], add=True)
    return carry

  lax.fori_loop(0, NCHUNK, chunk, 0)
  plsc.subcore_barrier()
  _flush_spmem(c, s, agg_sh, out_agg)


@functools.partial(
    pl.kernel, mesh=_MESH,
    out_type=jax.ShapeDtypeStruct((NC, N, D), _f32),
    scratch_types=[
        pltpu.VMEM((K,), jnp.int32),
        pltpu.VMEM((K, D), _f32),
        pltpu.VMEM_SHARED((N, D), _f32),
        pltpu.SemaphoreType.DMA,
    ])
def _sc_deg(dst_hbm, z_nd, ones_hbm, out_deg, didx, ones_v, deg_sh, sem):
  c = lax.axis_index("c")
  s = lax.axis_index("s")
  wid = c * NS + s

  _zero_spmem(s, z_nd, deg_sh)
  pltpu.sync_copy(ones_hbm, ones_v)
  plsc.subcore_barrier()

  ebase = wid * EPW

  def chunk(j, carry):
    base = ebase + j * K
    pltpu.sync_copy(dst_hbm.at[pl.ds(base, K)], didx)
    pltpu.sync_copy(ones_v, deg_sh.at[didx], add=True)
    return carry

  lax.fori_loop(0, NCHUNK, chunk, 0)
  plsc.subcore_barrier()
  _flush_spmem(c, s, deg_sh, out_deg)


def _gelu(u):
  return u * 0.5 * (1.0 + lax.erf(u * (2.0 ** -0.5)))


def _dense_layer_body(add_prev, aggp, degp, h, Wl, Wr, b, g, be, al, out):
  deg = jnp.maximum(degp[0, :, 0:1] + degp[1, :, 0:1], 1.0)
  agg = (aggp[0, :, :] + aggp[1, :, :]) / deg
  hv = h[...]
  u = (jnp.dot(agg, Wl[...], preferred_element_type=_f32) + b[...]
       + jnp.dot(hv, Wr[...], preferred_element_type=_f32))
  u = _gelu(u)
  mu = jnp.mean(u, axis=0, keepdims=True)
  sub = u - al[...] * mu
  var = jnp.mean(sub * sub, axis=0, keepdims=True)
  res = g[...] * sub * lax.rsqrt(var + 1e-5) + be[...]
  if add_prev:
    res = res + hv
  out[...] = res


def _make_dense_layer(add_prev):
  return pl.pallas_call(
      functools.partial(_dense_layer_body, add_prev),
      out_shape=jax.ShapeDtypeStruct((N, D), _f32),
  )


_dense0 = _make_dense_layer(False)
_dense_res = _make_dense_layer(True)


def _tail_body(o0, o1, o2, x, dyt_a, dyt_g, dyt_b, w1, b1, w2, b2, w3, b3,
               jw1, jb1, jw2, jb2, z_out, jv_out):
  a = dyt_a[0, 0]
  acc = b1[...]
  for i, o in enumerate((o0, o1, o2)):
    t = dyt_g[i:i + 1, :] * jnp.tanh(a * o[...]) + dyt_b[i:i + 1, :]
    acc = acc + jnp.dot(t, w1[i], preferred_element_type=_f32)
  z = _gelu(acc)
  z = _gelu(jnp.dot(z, w2[...], preferred_element_type=_f32) + b2[...])
  z = jnp.dot(z, w3[...], preferred_element_type=_f32) + b3[...] + x[...]
  nrm = jnp.sqrt(jnp.sum(z * z, axis=1, keepdims=True))
  z = z / (nrm + 1e-10)
  z_out[...] = z
  jm = jnp.mean(z, axis=0, keepdims=True)
  jv = _gelu(jnp.dot(jm, jw1[...], preferred_element_type=_f32) + jb1[...])
  jv = jnp.dot(jv, jw2[...], preferred_element_type=_f32) + jb2[...]
  jn = jnp.sqrt(jnp.sum(jv * jv, axis=1, keepdims=True))
  jv_out[...] = jv / (jn + 1e-10)


_tail = pl.pallas_call(
    _tail_body,
    out_shape=(jax.ShapeDtypeStruct((N, D), _f32),
               jax.ShapeDtypeStruct((1, D), _f32)),
)


def kernel(x, edge_index, sage_Wl, sage_Wr, sage_b, gn_gamma, gn_beta,
           gn_alpha, dyt_alpha, dyt_gamma, dyt_beta, lin1_W, lin1_b, lin2_W,
           lin2_b, lin3_W, lin3_b, jv1_W, jv1_b, jv2_W, jv2_b):
  ei = edge_index.astype(jnp.int32)
  src_i = ei[0]
  dst_i = ei[1]
  z_nd = jnp.zeros((N, D), _f32)
  ones_kd = jnp.ones((K, D), _f32)

  degp = _sc_deg(dst_i, z_nd, ones_kd)
  h = x
  outs = []
  for i in range(L):
    aggp = _sc_agg(h, src_i, dst_i, z_nd)
    dense = _dense0 if i == 0 else _dense_res
    h = dense(aggp, degp, h, sage_Wl[i], sage_Wr[i],
              sage_b[i].reshape(1, D), gn_gamma[i].reshape(1, D),
              gn_beta[i].reshape(1, D), gn_alpha[i].reshape(1, D))
    outs.append(h)

  z, jv = _tail(outs[0], outs[1], outs[2], x,
                dyt_alpha.reshape(1, 1), dyt_gamma.reshape(L, D),
                dyt_beta.reshape(L, D), lin1_W.reshape(L, D, D),
                lin1_b.reshape(1, D), lin2_W, lin2_b.reshape(1, D),
                lin3_W, lin3_b.reshape(1, D), jv1_W, jv1_b.reshape(1, D),
                jv2_W, jv2_b.reshape(1, D))
  return (z, jv)


# speedup vs baseline: 7.0099x; 1.4927x over previous
import functools

import jax
import jax.numpy as jnp
from jax import lax
from jax.experimental import pallas as pl
from jax.experimental.pallas import tpu as pltpu
from jax.experimental.pallas import tpu_sc as plsc

N = 10000
E = 320000
D = 128
L = 3

NC = 2
NS = 16
NW = NC * NS
EPW = E // NW
K = 80
NCHUNK = EPW // K
NB = 2
NOUT = (NCHUNK + NB - 1) // NB
RPT = 624
RTAIL = N - NS * RPT

_f32 = jnp.float32

_MESH = plsc.VectorSubcoreMesh(core_axis_name="c", subcore_axis_name="s")


def _zero_spmem(s, z_nd, sh):
  pltpu.sync_copy(z_nd.at[pl.ds(s * RPT, RPT)], sh.at[pl.ds(s * RPT, RPT)])

  @pl.when(s == 0)
  def _():
    pltpu.sync_copy(z_nd.at[pl.ds(NS * RPT, RTAIL)],
                    sh.at[pl.ds(NS * RPT, RTAIL)])


def _flush_spmem(c, s, sh, out):
  pltpu.sync_copy(sh.at[pl.ds(s * RPT, RPT)], out.at[c, pl.ds(s * RPT, RPT)])

  @pl.when(s == 0)
  def _():
    pltpu.sync_copy(sh.at[pl.ds(NS * RPT, RTAIL)],
                    out.at[c, pl.ds(NS * RPT, RTAIL)])


@functools.partial(
    pl.kernel, mesh=_MESH,
    out_type=jax.ShapeDtypeStruct((NC, N, D), _f32),
    scratch_types=[
        pltpu.VMEM((K,), jnp.int32),
        pltpu.VMEM((K,), jnp.int32),
        pltpu.VMEM((K,), jnp.int32),
        pltpu.VMEM((K,), jnp.int32),
        pltpu.VMEM((K, D), _f32),
        pltpu.VMEM((K, D), _f32),
        pltpu.VMEM_SHARED((N, D), _f32),
        pltpu.SemaphoreType.DMA,
        pltpu.SemaphoreType.DMA,
    ])
def _sc_agg(h_hbm, src_hbm, dst_hbm, z_nd, out_agg,
            sb0, sb1, db0, db1, r0, r1, agg_sh, sem0, sem1):
  c = lax.axis_index("c")
  s = lax.axis_index("s")
  wid = c * NS + s

  _zero_spmem(s, z_nd, agg_sh)
  plsc.subcore_barrier()
  ebase = wid * EPW

  sbufs = (sb0, sb1)
  dbufs = (db0, db1)
  rbufs = (r0, r1)
  sems = (sem0, sem1)

  for b in range(NB):
    base = ebase + b * K
    pltpu.sync_copy(src_hbm.at[pl.ds(base, K)], sbufs[b])
    pltpu.sync_copy(dst_hbm.at[pl.ds(base, K)], dbufs[b])
    pltpu.async_copy(h_hbm.at[sbufs[b]], rbufs[b], sems[b])

  def outer(i, carry):
    g0 = i * NB
    for b in range(NB):
      g = g0 + b

      @pl.when(g < NCHUNK)
      def _():
        pltpu.make_async_copy(h_hbm.at[sbufs[b]], rbufs[b], sems[b]).wait()
        pltpu.sync_copy(rbufs[b], agg_sh.at[dbufs[b]], add=True)
        nxt = g + NB

        @pl.when(nxt < NCHUNK)
        def _():
          nbase = ebase + nxt * K
          pltpu.sync_copy(src_hbm.at[pl.ds(nbase, K)], sbufs[b])
          pltpu.sync_copy(dst_hbm.at[pl.ds(nbase, K)], dbufs[b])
          pltpu.async_copy(h_hbm.at[sbufs[b]], rbufs[b], sems[b])

    return carry

  lax.fori_loop(0, NOUT, outer, 0)
  plsc.subcore_barrier()
  _flush_spmem(c, s, agg_sh, out_agg)


@functools.partial(
    pl.kernel, mesh=_MESH,
    out_type=jax.ShapeDtypeStruct((NC, N, D), _f32),
    scratch_types=[
        pltpu.VMEM((K,), jnp.int32),
        pltpu.VMEM((K, D), _f32),
        pltpu.VMEM_SHARED((N, D), _f32),
        pltpu.SemaphoreType.DMA,
    ])
def _sc_deg(dst_hbm, z_nd, ones_hbm, out_deg, didx, ones_v, deg_sh, sem):
  c = lax.axis_index("c")
  s = lax.axis_index("s")
  wid = c * NS + s

  _zero_spmem(s, z_nd, deg_sh)
  pltpu.sync_copy(ones_hbm, ones_v)
  plsc.subcore_barrier()

  ebase = wid * EPW

  def chunk(j, carry):
    pltpu.sync_copy(dst_hbm.at[pl.ds(ebase + j * K, K)], didx)
    pltpu.sync_copy(ones_v, deg_sh.at[didx], add=True)
    return carry

  lax.fori_loop(0, NCHUNK, chunk, 0)
  plsc.subcore_barrier()
  _flush_spmem(c, s, deg_sh, out_deg)


def _gelu(u):
  return u * 0.5 * (1.0 + lax.erf(u * (2.0 ** -0.5)))


def _dense_layer_body(add_prev, aggp, degp, h, Wl, Wr, b, g, be, al, out):
  deg = jnp.maximum(degp[0, :, 0:1] + degp[1, :, 0:1], 1.0)
  agg = (aggp[0, :, :] + aggp[1, :, :]) / deg
  hv = h[...]
  u = (jnp.dot(agg, Wl[...], preferred_element_type=_f32) + b[...]
       + jnp.dot(hv, Wr[...], preferred_element_type=_f32))
  u = _gelu(u)
  mu = jnp.mean(u, axis=0, keepdims=True)
  sub = u - al[...] * mu
  var = jnp.mean(sub * sub, axis=0, keepdims=True)
  res = g[...] * sub * lax.rsqrt(var + 1e-5) + be[...]
  if add_prev:
    res = res + hv
  out[...] = res


def _make_dense_layer(add_prev):
  return pl.pallas_call(
      functools.partial(_dense_layer_body, add_prev),
      out_shape=jax.ShapeDtypeStruct((N, D), _f32),
  )


_dense0 = _make_dense_layer(False)
_dense_res = _make_dense_layer(True)


def _tail_body(o0, o1, o2, x, dyt_a, dyt_g, dyt_b, w1, b1, w2, b2, w3, b3,
               jw1, jb1, jw2, jb2, z_out, jv_out):
  a = dyt_a[0, 0]
  acc = b1[...]
  for i, o in enumerate((o0, o1, o2)):
    t = dyt_g[i:i + 1, :] * jnp.tanh(a * o[...]) + dyt_b[i:i + 1, :]
    acc = acc + jnp.dot(t, w1[i], preferred_element_type=_f32)
  z = _gelu(acc)
  z = _gelu(jnp.dot(z, w2[...], preferred_element_type=_f32) + b2[...])
  z = jnp.dot(z, w3[...], preferred_element_type=_f32) + b3[...] + x[...]
  nrm = jnp.sqrt(jnp.sum(z * z, axis=1, keepdims=True))
  z = z / (nrm + 1e-10)
  z_out[...] = z
  jm = jnp.mean(z, axis=0, keepdims=True)
  jv = _gelu(jnp.dot(jm, jw1[...], preferred_element_type=_f32) + jb1[...])
  jv = jnp.dot(jv, jw2[...], preferred_element_type=_f32) + jb2[...]
  jn = jnp.sqrt(jnp.sum(jv * jv, axis=1, keepdims=True))
  jv_out[...] = jv / (jn + 1e-10)


_tail = pl.pallas_call(
    _tail_body,
    out_shape=(jax.ShapeDtypeStruct((N, D), _f32),
               jax.ShapeDtypeStruct((1, D), _f32)),
)


def kernel(x, edge_index, sage_Wl, sage_Wr, sage_b, gn_gamma, gn_beta,
           gn_alpha, dyt_alpha, dyt_gamma, dyt_beta, lin1_W, lin1_b, lin2_W,
           lin2_b, lin3_W, lin3_b, jv1_W, jv1_b, jv2_W, jv2_b):
  ei = edge_index.astype(jnp.int32)
  src_i = ei[0]
  dst_i = ei[1]
  z_nd = jnp.zeros((N, D), _f32)
  ones_kd = jnp.ones((K, D), _f32)

  degp = _sc_deg(dst_i, z_nd, ones_kd)
  h = x
  outs = []
  for i in range(L):
    aggp = _sc_agg(h, src_i, dst_i, z_nd)
    dense = _dense0 if i == 0 else _dense_res
    h = dense(aggp, degp, h, sage_Wl[i], sage_Wr[i],
              sage_b[i].reshape(1, D), gn_gamma[i].reshape(1, D),
              gn_beta[i].reshape(1, D), gn_alpha[i].reshape(1, D))
    outs.append(h)

  z, jv = _tail(outs[0], outs[1], outs[2], x,
                dyt_alpha.reshape(1, 1), dyt_gamma.reshape(L, D),
                dyt_beta.reshape(L, D), lin1_W.reshape(L, D, D),
                lin1_b.reshape(1, D), lin2_W, lin2_b.reshape(1, D),
                lin3_W, lin3_b.reshape(1, D), jv1_W, jv1_b.reshape(1, D),
                jv2_W, jv2_b.reshape(1, D))
  return (z, jv)


# speedup vs baseline: 7.0215x; 1.0017x over previous
import functools

import jax
import jax.numpy as jnp
from jax import lax
from jax.experimental import pallas as pl
from jax.experimental.pallas import tpu as pltpu
from jax.experimental.pallas import tpu_sc as plsc

N = 10000
E = 320000
D = 128
L = 3

NC = 2
NS = 16
NW = NC * NS
EPW = E // NW
K = 80
NCHUNK = EPW // K
NB = 4
NOUT = (NCHUNK + NB - 1) // NB
RPT = 624
RTAIL = N - NS * RPT

_f32 = jnp.float32

_MESH = plsc.VectorSubcoreMesh(core_axis_name="c", subcore_axis_name="s")


def _zero_spmem(s, z_nd, sh):
  pltpu.sync_copy(z_nd.at[pl.ds(s * RPT, RPT)], sh.at[pl.ds(s * RPT, RPT)])

  @pl.when(s == 0)
  def _():
    pltpu.sync_copy(z_nd.at[pl.ds(NS * RPT, RTAIL)],
                    sh.at[pl.ds(NS * RPT, RTAIL)])


def _flush_spmem(c, s, sh, out):
  pltpu.sync_copy(sh.at[pl.ds(s * RPT, RPT)], out.at[c, pl.ds(s * RPT, RPT)])

  @pl.when(s == 0)
  def _():
    pltpu.sync_copy(sh.at[pl.ds(NS * RPT, RTAIL)],
                    out.at[c, pl.ds(NS * RPT, RTAIL)])


@functools.partial(
    pl.kernel, mesh=_MESH,
    out_type=jax.ShapeDtypeStruct((NC, N, D), _f32),
    scratch_types=(
        [pltpu.VMEM((K,), jnp.int32)] * NB
        + [pltpu.VMEM((K,), jnp.int32)] * NB
        + [pltpu.VMEM((K, D), _f32)] * NB
        + [pltpu.VMEM_SHARED((N, D), _f32)]
        + [pltpu.SemaphoreType.DMA] * NB
    ))
def _sc_agg(h_hbm, src_hbm, dst_hbm, z_nd, out_agg, *scr):
  sbufs = scr[0:NB]
  dbufs = scr[NB:2 * NB]
  rbufs = scr[2 * NB:3 * NB]
  agg_sh = scr[3 * NB]
  sems = scr[3 * NB + 1:]

  c = lax.axis_index("c")
  s = lax.axis_index("s")
  wid = c * NS + s

  _zero_spmem(s, z_nd, agg_sh)
  plsc.subcore_barrier()
  ebase = wid * EPW

  for b in range(NB):
    base = ebase + b * K
    pltpu.sync_copy(src_hbm.at[pl.ds(base, K)], sbufs[b])
    pltpu.sync_copy(dst_hbm.at[pl.ds(base, K)], dbufs[b])
    pltpu.async_copy(h_hbm.at[sbufs[b]], rbufs[b], sems[b])

  def outer(i, carry):
    g0 = i * NB
    for b in range(NB):
      g = g0 + b

      @pl.when(g < NCHUNK)
      def _():
        pltpu.make_async_copy(h_hbm.at[sbufs[b]], rbufs[b], sems[b]).wait()
        pltpu.sync_copy(rbufs[b], agg_sh.at[dbufs[b]], add=True)
        nxt = g + NB

        @pl.when(nxt < NCHUNK)
        def _():
          nbase = ebase + nxt * K
          pltpu.sync_copy(src_hbm.at[pl.ds(nbase, K)], sbufs[b])
          pltpu.sync_copy(dst_hbm.at[pl.ds(nbase, K)], dbufs[b])
          pltpu.async_copy(h_hbm.at[sbufs[b]], rbufs[b], sems[b])

    return carry

  lax.fori_loop(0, NOUT, outer, 0)
  plsc.subcore_barrier()
  _flush_spmem(c, s, agg_sh, out_agg)


@functools.partial(
    pl.kernel, mesh=_MESH,
    out_type=jax.ShapeDtypeStruct((NC, N, D), _f32),
    scratch_types=[
        pltpu.VMEM((K,), jnp.int32),
        pltpu.VMEM((K, D), _f32),
        pltpu.VMEM_SHARED((N, D), _f32),
        pltpu.SemaphoreType.DMA,
    ])
def _sc_deg(dst_hbm, z_nd, ones_hbm, out_deg, didx, ones_v, deg_sh, sem):
  c = lax.axis_index("c")
  s = lax.axis_index("s")
  wid = c * NS + s

  _zero_spmem(s, z_nd, deg_sh)
  pltpu.sync_copy(ones_hbm, ones_v)
  plsc.subcore_barrier()

  ebase = wid * EPW

  def chunk(j, carry):
    pltpu.sync_copy(dst_hbm.at[pl.ds(ebase + j * K, K)], didx)
    pltpu.sync_copy(ones_v, deg_sh.at[didx], add=True)
    return carry

  lax.fori_loop(0, NCHUNK, chunk, 0)
  plsc.subcore_barrier()
  _flush_spmem(c, s, deg_sh, out_deg)


def _gelu(u):
  return u * 0.5 * (1.0 + lax.erf(u * (2.0 ** -0.5)))


def _dense_layer_body(add_prev, aggp, degp, h, Wl, Wr, b, g, be, al, out):
  deg = jnp.maximum(degp[0, :, 0:1] + degp[1, :, 0:1], 1.0)
  agg = (aggp[0, :, :] + aggp[1, :, :]) / deg
  hv = h[...]
  u = (jnp.dot(agg, Wl[...], preferred_element_type=_f32) + b[...]
       + jnp.dot(hv, Wr[...], preferred_element_type=_f32))
  u = _gelu(u)
  mu = jnp.mean(u, axis=0, keepdims=True)
  sub = u - al[...] * mu
  var = jnp.mean(sub * sub, axis=0, keepdims=True)
  res = g[...] * sub * lax.rsqrt(var + 1e-5) + be[...]
  if add_prev:
    res = res + hv
  out[...] = res


def _make_dense_layer(add_prev):
  return pl.pallas_call(
      functools.partial(_dense_layer_body, add_prev),
      out_shape=jax.ShapeDtypeStruct((N, D), _f32),
  )


_dense0 = _make_dense_layer(False)
_dense_res = _make_dense_layer(True)


def _tail_body(o0, o1, o2, x, dyt_a, dyt_g, dyt_b, w1, b1, w2, b2, w3, b3,
               jw1, jb1, jw2, jb2, z_out, jv_out):
  a = dyt_a[0, 0]
  acc = b1[...]
  for i, o in enumerate((o0, o1, o2)):
    t = dyt_g[i:i + 1, :] * jnp.tanh(a * o[...]) + dyt_b[i:i + 1, :]
    acc = acc + jnp.dot(t, w1[i], preferred_element_type=_f32)
  z = _gelu(acc)
  z = _gelu(jnp.dot(z, w2[...], preferred_element_type=_f32) + b2[...])
  z = jnp.dot(z, w3[...], preferred_element_type=_f32) + b3[...] + x[...]
  nrm = jnp.sqrt(jnp.sum(z * z, axis=1, keepdims=True))
  z = z / (nrm + 1e-10)
  z_out[...] = z
  jm = jnp.mean(z, axis=0, keepdims=True)
  jv = _gelu(jnp.dot(jm, jw1[...], preferred_element_type=_f32) + jb1[...])
  jv = jnp.dot(jv, jw2[...], preferred_element_type=_f32) + jb2[...]
  jn = jnp.sqrt(jnp.sum(jv * jv, axis=1, keepdims=True))
  jv_out[...] = jv / (jn + 1e-10)


_tail = pl.pallas_call(
    _tail_body,
    out_shape=(jax.ShapeDtypeStruct((N, D), _f32),
               jax.ShapeDtypeStruct((1, D), _f32)),
)


def kernel(x, edge_index, sage_Wl, sage_Wr, sage_b, gn_gamma, gn_beta,
           gn_alpha, dyt_alpha, dyt_gamma, dyt_beta, lin1_W, lin1_b, lin2_W,
           lin2_b, lin3_W, lin3_b, jv1_W, jv1_b, jv2_W, jv2_b):
  ei = edge_index.astype(jnp.int32)
  src_i = ei[0]
  dst_i = ei[1]
  z_nd = jnp.zeros((N, D), _f32)
  ones_kd = jnp.ones((K, D), _f32)

  degp = _sc_deg(dst_i, z_nd, ones_kd)
  h = x
  outs = []
  for i in range(L):
    aggp = _sc_agg(h, src_i, dst_i, z_nd)
    dense = _dense0 if i == 0 else _dense_res
    h = dense(aggp, degp, h, sage_Wl[i], sage_Wr[i],
              sage_b[i].reshape(1, D), gn_gamma[i].reshape(1, D),
              gn_beta[i].reshape(1, D), gn_alpha[i].reshape(1, D))
    outs.append(h)

  z, jv = _tail(outs[0], outs[1], outs[2], x,
                dyt_alpha.reshape(1, 1), dyt_gamma.reshape(L, D),
                dyt_beta.reshape(L, D), lin1_W.reshape(L, D, D),
                lin1_b.reshape(1, D), lin2_W, lin2_b.reshape(1, D),
                lin3_W, lin3_b.reshape(1, D), jv1_W, jv1_b.reshape(1, D),
                jv2_W, jv2_b.reshape(1, D))
  return (z, jv)


# speedup vs baseline: 11.7961x; 1.6800x over previous
import functools

import jax
import jax.numpy as jnp
from jax import lax
from jax.experimental import pallas as pl
from jax.experimental.pallas import tpu as pltpu
from jax.experimental.pallas import tpu_sc as plsc

N = 10000
E = 320000
D = 128
L = 3

NC = 2
NS = 16
NW = NC * NS
EPW = E // NW
K = 80
NCHUNK = EPW // K
NB = 4
NOUT = (NCHUNK + NB - 1) // NB
RPT = 624
RTAIL = N - NS * RPT

_f32 = jnp.float32

_MESH = plsc.VectorSubcoreMesh(core_axis_name="c", subcore_axis_name="s")


def _zero_spmem(s, z_nd, sh):
  pltpu.sync_copy(z_nd.at[pl.ds(s * RPT, RPT)], sh.at[pl.ds(s * RPT, RPT)])

  @pl.when(s == 0)
  def _():
    pltpu.sync_copy(z_nd.at[pl.ds(NS * RPT, RTAIL)],
                    sh.at[pl.ds(NS * RPT, RTAIL)])


def _flush_spmem(c, s, sh, out):
  pltpu.sync_copy(sh.at[pl.ds(s * RPT, RPT)], out.at[c, pl.ds(s * RPT, RPT)])

  @pl.when(s == 0)
  def _():
    pltpu.sync_copy(sh.at[pl.ds(NS * RPT, RTAIL)],
                    out.at[c, pl.ds(NS * RPT, RTAIL)])


@functools.partial(
    pl.kernel, mesh=_MESH,
    out_type=jax.ShapeDtypeStruct((NC, N, D), _f32),
    scratch_types=(
        [pltpu.VMEM((K,), jnp.int32)] * NB
        + [pltpu.VMEM((K,), jnp.int32)] * NB
        + [pltpu.VMEM((K, D), _f32)] * NB
        + [pltpu.VMEM_SHARED((N, D), _f32)]
        + [pltpu.SemaphoreType.DMA] * NB
        + [pltpu.SemaphoreType.DMA] * NB
    ))
def _sc_agg(h_hbm, src_hbm, dst_hbm, z_nd, out_agg, *scr):
  sbufs = scr[0:NB]
  dbufs = scr[NB:2 * NB]
  rbufs = scr[2 * NB:3 * NB]
  agg_sh = scr[3 * NB]
  gsems = scr[3 * NB + 1:4 * NB + 1]
  isems = scr[4 * NB + 1:]

  c = lax.axis_index("c")
  s = lax.axis_index("s")
  wid = c * NS + s

  _zero_spmem(s, z_nd, agg_sh)
  plsc.subcore_barrier()
  ebase = wid * EPW

  def idx_start(g, b):
    base = ebase + g * K
    pltpu.async_copy(src_hbm.at[pl.ds(base, K)], sbufs[b], isems[b])
    pltpu.async_copy(dst_hbm.at[pl.ds(base, K)], dbufs[b], isems[b])

  def idx_wait_gather_start(g, b):
    base = ebase + g * K
    pltpu.make_async_copy(src_hbm.at[pl.ds(base, K)], sbufs[b],
                          isems[b]).wait()
    pltpu.make_async_copy(dst_hbm.at[pl.ds(base, K)], dbufs[b],
                          isems[b]).wait()
    pltpu.async_copy(h_hbm.at[sbufs[b]], rbufs[b], gsems[b])

  for b in range(NB):
    idx_start(b, b)
  for b in range(NB - 1):
    idx_wait_gather_start(b, b)

  def outer(i, carry):
    g0 = i * NB
    for b in range(NB):
      g = g0 + b

      @pl.when(g < NCHUNK)
      def _():
        pltpu.make_async_copy(h_hbm.at[sbufs[b]], rbufs[b], gsems[b]).wait()
        pltpu.sync_copy(rbufs[b], agg_sh.at[dbufs[b]], add=True)
        nxt = g + NB

        @pl.when(nxt < NCHUNK)
        def _():
          idx_start(nxt, b)

        g2 = g + NB - 1
        b2 = (b + NB - 1) % NB

        @pl.when(g2 < NCHUNK)
        def _():
          idx_wait_gather_start(g2, b2)

    return carry

  lax.fori_loop(0, NOUT, outer, 0)
  plsc.subcore_barrier()
  _flush_spmem(c, s, agg_sh, out_agg)


@functools.partial(
    pl.kernel, mesh=_MESH,
    out_type=jax.ShapeDtypeStruct((NC, N, D), _f32),
    scratch_types=(
        [pltpu.VMEM((K,), jnp.int32)] * NB
        + [pltpu.VMEM((K, D), _f32)]
        + [pltpu.VMEM_SHARED((N, D), _f32)]
        + [pltpu.SemaphoreType.DMA] * NB
    ))
def _sc_deg(dst_hbm, z_nd, ones_hbm, out_deg, *scr):
  dbufs = scr[0:NB]
  ones_v = scr[NB]
  deg_sh = scr[NB + 1]
  isems = scr[NB + 2:]

  c = lax.axis_index("c")
  s = lax.axis_index("s")
  wid = c * NS + s

  _zero_spmem(s, z_nd, deg_sh)
  pltpu.sync_copy(ones_hbm, ones_v)
  plsc.subcore_barrier()

  ebase = wid * EPW

  for b in range(NB):
    pltpu.async_copy(dst_hbm.at[pl.ds(ebase + b * K, K)], dbufs[b], isems[b])

  def outer(i, carry):
    g0 = i * NB
    for b in range(NB):
      g = g0 + b

      @pl.when(g < NCHUNK)
      def _():
        pltpu.make_async_copy(dst_hbm.at[pl.ds(ebase + g * K, K)], dbufs[b],
                              isems[b]).wait()
        pltpu.sync_copy(ones_v, deg_sh.at[dbufs[b]], add=True)
        nxt = g + NB

        @pl.when(nxt < NCHUNK)
        def _():
          pltpu.async_copy(dst_hbm.at[pl.ds(ebase + nxt * K, K)], dbufs[b],
                           isems[b])

    return carry

  lax.fori_loop(0, NOUT, outer, 0)
  plsc.subcore_barrier()
  _flush_spmem(c, s, deg_sh, out_deg)


def _gelu(u):
  return u * 0.5 * (1.0 + lax.erf(u * (2.0 ** -0.5)))


def _dense_layer_body(add_prev, aggp, degp, h, Wl, Wr, b, g, be, al, out):
  deg = jnp.maximum(degp[0, :, 0:1] + degp[1, :, 0:1], 1.0)
  agg = (aggp[0, :, :] + aggp[1, :, :]) / deg
  hv = h[...]
  u = (jnp.dot(agg, Wl[...], preferred_element_type=_f32) + b[...]
       + jnp.dot(hv, Wr[...], preferred_element_type=_f32))
  u = _gelu(u)
  mu = jnp.mean(u, axis=0, keepdims=True)
  sub = u - al[...] * mu
  var = jnp.mean(sub * sub, axis=0, keepdims=True)
  res = g[...] * sub * lax.rsqrt(var + 1e-5) + be[...]
  if add_prev:
    res = res + hv
  out[...] = res


def _make_dense_layer(add_prev):
  return pl.pallas_call(
      functools.partial(_dense_layer_body, add_prev),
      out_shape=jax.ShapeDtypeStruct((N, D), _f32),
  )


_dense0 = _make_dense_layer(False)
_dense_res = _make_dense_layer(True)


def _tail_body(o0, o1, o2, x, dyt_a, dyt_g, dyt_b, w1, b1, w2, b2, w3, b3,
               jw1, jb1, jw2, jb2, z_out, jv_out):
  a = dyt_a[0, 0]
  acc = b1[...]
  for i, o in enumerate((o0, o1, o2)):
    t = dyt_g[i:i + 1, :] * jnp.tanh(a * o[...]) + dyt_b[i:i + 1, :]
    acc = acc + jnp.dot(t, w1[i], preferred_element_type=_f32)
  z = _gelu(acc)
  z = _gelu(jnp.dot(z, w2[...], preferred_element_type=_f32) + b2[...])
  z = jnp.dot(z, w3[...], preferred_element_type=_f32) + b3[...] + x[...]
  nrm = jnp.sqrt(jnp.sum(z * z, axis=1, keepdims=True))
  z = z / (nrm + 1e-10)
  z_out[...] = z
  jm = jnp.mean(z, axis=0, keepdims=True)
  jv = _gelu(jnp.dot(jm, jw1[...], preferred_element_type=_f32) + jb1[...])
  jv = jnp.dot(jv, jw2[...], preferred_element_type=_f32) + jb2[...]
  jn = jnp.sqrt(jnp.sum(jv * jv, axis=1, keepdims=True))
  jv_out[...] = jv / (jn + 1e-10)


_tail = pl.pallas_call(
    _tail_body,
    out_shape=(jax.ShapeDtypeStruct((N, D), _f32),
               jax.ShapeDtypeStruct((1, D), _f32)),
)


def kernel(x, edge_index, sage_Wl, sage_Wr, sage_b, gn_gamma, gn_beta,
           gn_alpha, dyt_alpha, dyt_gamma, dyt_beta, lin1_W, lin1_b, lin2_W,
           lin2_b, lin3_W, lin3_b, jv1_W, jv1_b, jv2_W, jv2_b):
  ei = edge_index.astype(jnp.int32)
  src_i = ei[0]
  dst_i = ei[1]
  z_nd = jnp.zeros((N, D), _f32)
  ones_kd = jnp.ones((K, D), _f32)

  degp = _sc_deg(dst_i, z_nd, ones_kd)
  h = x
  outs = []
  for i in range(L):
    aggp = _sc_agg(h, src_i, dst_i, z_nd)
    dense = _dense0 if i == 0 else _dense_res
    h = dense(aggp, degp, h, sage_Wl[i], sage_Wr[i],
              sage_b[i].reshape(1, D), gn_gamma[i].reshape(1, D),
              gn_beta[i].reshape(1, D), gn_alpha[i].reshape(1, D))
    outs.append(h)

  z, jv = _tail(outs[0], outs[1], outs[2], x,
                dyt_alpha.reshape(1, 1), dyt_gamma.reshape(L, D),
                dyt_beta.reshape(L, D), lin1_W.reshape(L, D, D),
                lin1_b.reshape(1, D), lin2_W, lin2_b.reshape(1, D),
                lin3_W, lin3_b.reshape(1, D), jv1_W, jv1_b.reshape(1, D),
                jv2_W, jv2_b.reshape(1, D))
  return (z, jv)


# speedup vs baseline: 11.8717x; 1.0064x over previous
import functools

import jax
import jax.numpy as jnp
from jax import lax
from jax.experimental import pallas as pl
from jax.experimental.pallas import tpu as pltpu
from jax.experimental.pallas import tpu_sc as plsc

N = 10000
E = 320000
D = 128
L = 3

NC = 2
NS = 16
NW = NC * NS
EPW = E // NW
K = 80
NCHUNK = EPW // K
NB = 4
NOUT = (NCHUNK + NB - 1) // NB
RPT = 624
RTAIL = N - NS * RPT
NR = 80
NPAD = NR * D

_f32 = jnp.float32

_MESH = plsc.VectorSubcoreMesh(core_axis_name="c", subcore_axis_name="s")


def _zero_spmem(s, z_nd, sh):
  pltpu.sync_copy(z_nd.at[pl.ds(s * RPT, RPT)], sh.at[pl.ds(s * RPT, RPT)])

  @pl.when(s == 0)
  def _():
    pltpu.sync_copy(z_nd.at[pl.ds(NS * RPT, RTAIL)],
                    sh.at[pl.ds(NS * RPT, RTAIL)])


def _flush_spmem(c, s, sh, out):
  pltpu.sync_copy(sh.at[pl.ds(s * RPT, RPT)], out.at[c, pl.ds(s * RPT, RPT)])

  @pl.when(s == 0)
  def _():
    pltpu.sync_copy(sh.at[pl.ds(NS * RPT, RTAIL)],
                    out.at[c, pl.ds(NS * RPT, RTAIL)])


@functools.partial(
    pl.kernel, mesh=_MESH,
    out_type=jax.ShapeDtypeStruct((NC, N, D), _f32),
    scratch_types=(
        [pltpu.VMEM((K,), jnp.int32)] * NB
        + [pltpu.VMEM((K,), jnp.int32)] * NB
        + [pltpu.VMEM((K, D), _f32)] * NB
        + [pltpu.VMEM_SHARED((N, D), _f32)]
        + [pltpu.SemaphoreType.DMA] * NB
        + [pltpu.SemaphoreType.DMA] * NB
    ))
def _sc_agg(h_hbm, src_hbm, dst_hbm, z_nd, out_agg, *scr):
  sbufs = scr[0:NB]
  dbufs = scr[NB:2 * NB]
  rbufs = scr[2 * NB:3 * NB]
  agg_sh = scr[3 * NB]
  gsems = scr[3 * NB + 1:4 * NB + 1]
  isems = scr[4 * NB + 1:]

  c = lax.axis_index("c")
  s = lax.axis_index("s")
  wid = c * NS + s

  _zero_spmem(s, z_nd, agg_sh)
  plsc.subcore_barrier()
  ebase = wid * EPW

  def idx_start(g, b):
    base = ebase + g * K
    pltpu.async_copy(src_hbm.at[pl.ds(base, K)], sbufs[b], isems[b])
    pltpu.async_copy(dst_hbm.at[pl.ds(base, K)], dbufs[b], isems[b])

  def idx_wait_gather_start(g, b):
    base = ebase + g * K
    pltpu.make_async_copy(src_hbm.at[pl.ds(base, K)], sbufs[b],
                          isems[b]).wait()
    pltpu.make_async_copy(dst_hbm.at[pl.ds(base, K)], dbufs[b],
                          isems[b]).wait()
    pltpu.async_copy(h_hbm.at[sbufs[b]], rbufs[b], gsems[b])

  for b in range(NB):
    idx_start(b, b)
  for b in range(NB - 1):
    idx_wait_gather_start(b, b)

  def outer(i, carry):
    g0 = i * NB
    for b in range(NB):
      g = g0 + b

      @pl.when(g < NCHUNK)
      def _():
        pltpu.make_async_copy(h_hbm.at[sbufs[b]], rbufs[b], gsems[b]).wait()
        pltpu.sync_copy(rbufs[b], agg_sh.at[dbufs[b]], add=True)
        nxt = g + NB

        @pl.when(nxt < NCHUNK)
        def _():
          idx_start(nxt, b)

        g2 = g + NB - 1
        b2 = (b + NB - 1) % NB

        @pl.when(g2 < NCHUNK)
        def _():
          idx_wait_gather_start(g2, b2)

    return carry

  lax.fori_loop(0, NOUT, outer, 0)
  plsc.subcore_barrier()
  _flush_spmem(c, s, agg_sh, out_agg)


@functools.partial(
    pl.kernel, mesh=_MESH,
    out_type=jax.ShapeDtypeStruct((NC, N, D), _f32),
    scratch_types=(
        [pltpu.VMEM((K,), jnp.int32)] * NB
        + [pltpu.VMEM((K, D), _f32)]
        + [pltpu.VMEM_SHARED((N, D), _f32)]
        + [pltpu.SemaphoreType.DMA] * NB
    ))
def _sc_deg(dst_hbm, z_nd, ones_hbm, out_deg, *scr):
  dbufs = scr[0:NB]
  ones_v = scr[NB]
  deg_sh = scr[NB + 1]
  isems = scr[NB + 2:]

  c = lax.axis_index("c")
  s = lax.axis_index("s")
  wid = c * NS + s

  _zero_spmem(s, z_nd, deg_sh)
  pltpu.sync_copy(ones_hbm, ones_v)
  plsc.subcore_barrier()

  ebase = wid * EPW

  for b in range(NB):
    pltpu.async_copy(dst_hbm.at[pl.ds(ebase + b * K, K)], dbufs[b], isems[b])

  def outer(i, carry):
    g0 = i * NB
    for b in range(NB):
      g = g0 + b

      @pl.when(g < NCHUNK)
      def _():
        pltpu.make_async_copy(dst_hbm.at[pl.ds(ebase + g * K, K)], dbufs[b],
                              isems[b]).wait()
        pltpu.sync_copy(ones_v, deg_sh.at[dbufs[b]], add=True)
        nxt = g + NB

        @pl.when(nxt < NCHUNK)
        def _():
          pltpu.async_copy(dst_hbm.at[pl.ds(ebase + nxt * K, K)], dbufs[b],
                           isems[b])

    return carry

  lax.fori_loop(0, NOUT, outer, 0)
  plsc.subcore_barrier()
  _flush_spmem(c, s, deg_sh, out_deg)


def _gelu(u):
  return u * 0.5 * (1.0 + lax.erf(u * (2.0 ** -0.5)))


def _dense_pre_body(h, Wr, b, out):
  out[...] = jnp.dot(h[...], Wr[...], preferred_element_type=_f32) + b[...]


_dense_pre = pl.pallas_call(
    _dense_pre_body,
    out_shape=jax.ShapeDtypeStruct((N, D), _f32),
)


def _deg_col_body(degp, out):
  out[...] = 1.0 / jnp.maximum(degp[0, :, 0:1] + degp[1, :, 0:1], 1.0)


_deg_col = pl.pallas_call(
    _deg_col_body,
    out_shape=jax.ShapeDtypeStruct((N, 1), _f32),
)


def _layer_post(add_prev, aggp, rdeg, hwr, h, Wl, g, be, al):
  agg = (aggp[0, :, :] + aggp[1, :, :]) * rdeg[...]
  u = jnp.dot(agg, Wl[...], preferred_element_type=_f32) + hwr[...]
  u = _gelu(u)
  mu = jnp.mean(u, axis=0, keepdims=True)
  sub = u - al[...] * mu
  var = jnp.mean(sub * sub, axis=0, keepdims=True)
  res = g[...] * sub * lax.rsqrt(var + 1e-5) + be[...]
  if add_prev:
    res = res + h[...]
  return res


def _dense_fin_body(add_prev, aggp, rdeg, hwr, h, Wl, g, be, al, out):
  out[...] = _layer_post(add_prev, aggp, rdeg, hwr, h, Wl, g, be, al)


def _make_dense_fin(add_prev):
  return pl.pallas_call(
      functools.partial(_dense_fin_body, add_prev),
      out_shape=jax.ShapeDtypeStruct((N, D), _f32),
  )


_dense_fin0 = _make_dense_fin(False)
_dense_fin_res = _make_dense_fin(True)


def _tail_body(aggp, rdeg, hwr, h, Wl, g, be, al, o0, o1, x, dyt_a, dyt_g,
               dyt_b, w1, b1, w2, b2, w3, b3, jw1, jb1, jw2, jb2,
               z_out, jv_out):
  o2 = _layer_post(True, aggp, rdeg, hwr, h, Wl, g, be, al)
  a = dyt_a[0, 0]
  acc = b1[...]
  for i, o in enumerate((o0[...], o1[...], o2)):
    t = dyt_g[i:i + 1, :] * jnp.tanh(a * o) + dyt_b[i:i + 1, :]
    acc = acc + jnp.dot(t, w1[i], preferred_element_type=_f32)
  z = _gelu(acc)
  z = _gelu(jnp.dot(z, w2[...], preferred_element_type=_f32) + b2[...])
  z = jnp.dot(z, w3[...], preferred_element_type=_f32) + b3[...] + x[...]
  nrm = jnp.sqrt(jnp.sum(z * z, axis=1, keepdims=True))
  z = z / (nrm + 1e-10)
  z_out[...] = z
  jm = jnp.mean(z, axis=0, keepdims=True)
  jv = _gelu(jnp.dot(jm, jw1[...], preferred_element_type=_f32) + jb1[...])
  jv = jnp.dot(jv, jw2[...], preferred_element_type=_f32) + jb2[...]
  jn = jnp.sqrt(jnp.sum(jv * jv, axis=1, keepdims=True))
  jv_out[...] = jv / (jn + 1e-10)


_tail = pl.pallas_call(
    _tail_body,
    out_shape=(jax.ShapeDtypeStruct((N, D), _f32),
               jax.ShapeDtypeStruct((1, D), _f32)),
)


def kernel(x, edge_index, sage_Wl, sage_Wr, sage_b, gn_gamma, gn_beta,
           gn_alpha, dyt_alpha, dyt_gamma, dyt_beta, lin1_W, lin1_b, lin2_W,
           lin2_b, lin3_W, lin3_b, jv1_W, jv1_b, jv2_W, jv2_b):
  ei = edge_index.astype(jnp.int32)
  src_i = ei[0]
  dst_i = ei[1]
  z_nd = jnp.zeros((N, D), _f32)
  ones_kd = jnp.ones((K, D), _f32)

  degp = _sc_deg(dst_i, z_nd, ones_kd)
  rdeg = _deg_col(degp)
  h = x
  outs = []
  for i in range(L - 1):
    aggp = _sc_agg(h, src_i, dst_i, z_nd)
    hwr = _dense_pre(h, sage_Wr[i], sage_b[i].reshape(1, D))
    dense = _dense_fin0 if i == 0 else _dense_fin_res
    h = dense(aggp, rdeg, hwr, h, sage_Wl[i], gn_gamma[i].reshape(1, D),
              gn_beta[i].reshape(1, D), gn_alpha[i].reshape(1, D))
    outs.append(h)

  aggp = _sc_agg(h, src_i, dst_i, z_nd)
  hwr = _dense_pre(h, sage_Wr[2], sage_b[2].reshape(1, D))
  z, jv = _tail(aggp, rdeg, hwr, h, sage_Wl[2], gn_gamma[2].reshape(1, D),
                gn_beta[2].reshape(1, D), gn_alpha[2].reshape(1, D),
                outs[0], outs[1], x,
                dyt_alpha.reshape(1, 1), dyt_gamma.reshape(L, D),
                dyt_beta.reshape(L, D), lin1_W.reshape(L, D, D),
                lin1_b.reshape(1, D), lin2_W, lin2_b.reshape(1, D),
                lin3_W, lin3_b.reshape(1, D), jv1_W, jv1_b.reshape(1, D),
                jv2_W, jv2_b.reshape(1, D))
  return (z, jv)
